# baseline (speedup 1.0000x reference)
import functools

import jax
import jax.numpy as jnp
from jax import lax
from jax.experimental import pallas as pl
from jax.experimental.pallas import tpu as pltpu
from jax.experimental.pallas import tpu_sc as plsc

VOCAB = 1000000
HIDDEN = 64
EPS = 1e-5

NW = 32
CHUNK = 128


def _rsqrt16(x):
    xi = plsc.bitcast(x, jnp.int32)
    yi = jnp.int32(0x5F3759DF) - (xi >> 1)
    y = plsc.bitcast(yi, jnp.float32)
    half_x = 0.5 * x
    for _ in range(3):
        y = y * (1.5 - half_x * y * y)
    return y


def _ln_body(x_hbm, tbl_hbm, w_hbm, b_hbm, out_hbm, idx_v, rows_v, w_v, b_v, sem,
             *, nchunk):
    cid = lax.axis_index("c")
    sid = lax.axis_index("s")
    wid = sid * 2 + cid

    pltpu.sync_copy(x_hbm.at[wid], idx_v)
    pltpu.sync_copy(w_hbm, w_v)
    pltpu.sync_copy(b_hbm, b_v)

    lane = lax.iota(jnp.int32, 16)
    inv_h = jnp.float32(1.0 / HIDDEN)

    def chunk_body(i, carry):
        pltpu.async_copy(tbl_hbm.at[idx_v.at[i]], rows_v, sem).wait()

        def grp_body(g, c2):
            rows16 = g * 16 + lane
            wvecs = [w_v[pl.ds(k * 16, 16)] for k in range(HIDDEN // 16)]
            bvecs = [b_v[pl.ds(k * 16, 16)] for k in range(HIDDEN // 16)]
            s = jnp.zeros((16,), jnp.float32)
            q = jnp.zeros((16,), jnp.float32)
            for j in range(HIDDEN):
                cj = jnp.full((16,), j, jnp.int32)
                v = plsc.load_gather(rows_v, [rows16, cj])
                s = s + v
                q = q + v * v
            mean = s * inv_h
            var = q * inv_h - mean * mean
            inv = _rsqrt16(var + EPS)
            for j in range(HIDDEN):
                cj = jnp.full((16,), j, jnp.int32)
                v = plsc.load_gather(rows_v, [rows16, cj])
                scale = inv * wvecs[j // 16][j % 16]
                shift = bvecs[j // 16][j % 16] - mean * scale
                plsc.store_scatter(rows_v, [rows16, cj], v * scale + shift)
            return c2

        lax.fori_loop(0, CHUNK // 16, grp_body, 0)
        pltpu.sync_copy(rows_v, out_hbm.at[wid, i])
        return carry

    lax.fori_loop(0, nchunk, chunk_body, 0)


def kernel(x, word_table, ln_weight, ln_bias):
    rows, cols = x.shape
    total = rows * cols
    nchunk = total // (NW * CHUNK)
    xf = x.reshape(NW, nchunk, CHUNK).astype(jnp.int32)

    mesh = plsc.VectorSubcoreMesh(core_axis_name="c", subcore_axis_name="s")
    run = pl.kernel(
        functools.partial(_ln_body, nchunk=nchunk),
        mesh=mesh,
        compiler_params=pltpu.CompilerParams(
            needs_layout_passes=False, use_tc_tiling_on_sc=False),
        out_type=jax.ShapeDtypeStruct((NW, nchunk, CHUNK, HIDDEN), jnp.float32),
        scratch_types=[
            pltpu.VMEM((nchunk, CHUNK), jnp.int32),
            pltpu.VMEM((CHUNK, HIDDEN), jnp.float32),
            pltpu.VMEM((HIDDEN,), jnp.float32),
            pltpu.VMEM((HIDDEN,), jnp.float32),
            pltpu.SemaphoreType.DMA,
        ],
    )
    out = run(xf, word_table, ln_weight, ln_bias)
    return out.reshape(rows, cols, HIDDEN)


# speedup vs baseline: 2.6905x; 2.6905x over previous
import functools

import jax
import jax.numpy as jnp
from jax import lax
from jax.experimental import pallas as pl
from jax.experimental.pallas import tpu as pltpu
from jax.experimental.pallas import tpu_sc as plsc

VOCAB = 1000000
HIDDEN = 64
EPS = 1e-5

NW = 32
CHUNK = 128


def _rsqrt16(x):
    xi = plsc.bitcast(x, jnp.int32)
    yi = jnp.int32(0x5F3759DF) - (xi >> 1)
    y = plsc.bitcast(yi, jnp.float32)
    half_x = 0.5 * x
    for _ in range(3):
        y = y * (1.5 - half_x * y * y)
    return y


def _bcast(s):
    return lax.broadcast_in_dim(s, (16,), ())


def _ln_rows16(rows, base, wv, bv):
    inv_h = jnp.float32(1.0 / HIDDEN)
    for rr in range(16):
        r = base + rr
        v = [rows[r, pl.ds(k * 16, 16)] for k in range(HIDDEN // 16)]
        s = (v[0] + v[1]) + (v[2] + v[3])
        q = (v[0] * v[0] + v[1] * v[1]) + (v[2] * v[2] + v[3] * v[3])
        mean = _bcast(jnp.sum(s)) * inv_h
        var = _bcast(jnp.sum(q)) * inv_h - mean * mean
        inv = _rsqrt16(var + EPS)
        for k in range(HIDDEN // 16):
            scale = inv * wv[k]
            rows[r, pl.ds(k * 16, 16)] = (v[k] - mean) * scale + bv[k]


def _ln_body(x_hbm, tbl_hbm, w_hbm, b_hbm, out_hbm, idx_v, rows2, w_v, b_v,
             sems, *, nchunk):
    cid = lax.axis_index("c")
    sid = lax.axis_index("s")
    wid = sid * 2 + cid

    pltpu.sync_copy(x_hbm.at[wid], idx_v)
    pltpu.sync_copy(w_hbm, w_v)
    pltpu.sync_copy(b_hbm, b_v)

    wv = [w_v[pl.ds(k * 16, 16)] for k in range(HIDDEN // 16)]
    bv = [b_v[pl.ds(k * 16, 16)] for k in range(HIDDEN // 16)]

    def gather(i, b):
        pltpu.make_async_copy(
            tbl_hbm.at[idx_v.at[i]], rows2.at[b], sems.at[b]).start()

    def gather_wait(i, b):
        pltpu.make_async_copy(
            tbl_hbm.at[idx_v.at[i]], rows2.at[b], sems.at[b]).wait()

    gather(0, 0)
    gather(1, 1)

    def pair_body(p, carry):
        for b in range(2):
            i = 2 * p + b
            gather_wait(i, b)

            def grp_body(g, c2):
                _ln_rows16(rows2.at[b], g * 16, wv, bv)
                return c2

            lax.fori_loop(0, CHUNK // 16, grp_body, 0)
            pltpu.sync_copy(rows2.at[b], out_hbm.at[wid, i])

            @pl.when(i + 2 < nchunk)
            def _():
                gather(i + 2, b)
        return carry

    lax.fori_loop(0, nchunk // 2, pair_body, 0)


def kernel(x, word_table, ln_weight, ln_bias):
    rows, cols = x.shape
    total = rows * cols
    nchunk = total // (NW * CHUNK)
    xf = x.reshape(NW, nchunk, CHUNK).astype(jnp.int32)

    mesh = plsc.VectorSubcoreMesh(core_axis_name="c", subcore_axis_name="s")
    run = pl.kernel(
        functools.partial(_ln_body, nchunk=nchunk),
        mesh=mesh,
        compiler_params=pltpu.CompilerParams(
            needs_layout_passes=False, use_tc_tiling_on_sc=False),
        out_type=jax.ShapeDtypeStruct((NW, nchunk, CHUNK, HIDDEN), jnp.float32),
        scratch_types=[
            pltpu.VMEM((nchunk, CHUNK), jnp.int32),
            pltpu.VMEM((2, CHUNK, HIDDEN), jnp.float32),
            pltpu.VMEM((HIDDEN,), jnp.float32),
            pltpu.VMEM((HIDDEN,), jnp.float32),
            pltpu.SemaphoreType.DMA((2,)),
        ],
    )
    out = run(xf, word_table, ln_weight, ln_bias)
    return out.reshape(rows, cols, HIDDEN)
